# Initial kernel scaffold; baseline (speedup 1.0000x reference)
import jax
import jax.numpy as jnp
from jax.experimental import pallas as pl


def kernel(x, edge_index, batch, W1, b1, W2, b2, W3, b3):
    raise NotImplementedError("write your pallas kernel here")


# speedup vs baseline: 5.2332x; 5.2332x over previous
import functools

import jax
import jax.numpy as jnp
from jax import lax
from jax.experimental import pallas as pl
from jax.experimental.pallas import tpu as pltpu
from jax.experimental.pallas import tpu_sc as plsc

N = 10000
E = 320000
D = 128
H = 64
G = 64
OUT = 1

NC = 2
NS = 16
NW = NC * NS
EPT = E // NW
NH = 10240
EPTP = 10240
CG = 16
CW = H // CG
TB = N * CW
TBP = TB + 64
AW = (E // 2) * CW
CHW = 8192
NCH = 80
AWP = NCH * CHW
VPC = CHW // 16

RB = 1000
NBK = N // RB

_f32 = jnp.float32


def _mesh():
    return plsc.VectorSubcoreMesh(core_axis_name="c", subcore_axis_name="s")


_SC_PARAMS = pltpu.CompilerParams(needs_layout_passes=False)


def _sc_degree(dst2, zn):
    @functools.partial(
        pl.kernel,
        out_type=jax.ShapeDtypeStruct((NW, NH), _f32),
        mesh=_mesh(),
        scratch_types=[
            pltpu.VMEM((NH,), _f32),
            pltpu.VMEM((EPTP,), jnp.int32),
        ],
        compiler_params=_SC_PARAMS,
    )
    def deg_kernel(dst_hbm, zn_hbm, out, hist, dv):
        cid = lax.axis_index("c")
        sid = lax.axis_index("s")
        wid = cid * NS + sid
        pltpu.sync_copy(dst_hbm.at[wid], dv)
        pltpu.sync_copy(zn_hbm, hist)
        ones16 = jnp.full((16,), 1.0, _f32)

        def step(i, c):
            for u in range(4):
                idx = dv[pl.ds((i * 4 + u) * 16, 16)]
                plsc.addupdate_scatter(hist, [idx], ones16)
            return c

        lax.fori_loop(0, EPTP // 64, step, 0)
        pltpu.sync_copy(hist, out.at[wid])

    return deg_kernel(dst2, zn)


def _sc_edge(g_all, asrc, adst, zer):
    @functools.partial(
        pl.kernel,
        out_type=jax.ShapeDtypeStruct((CG, 2, 1, TBP), _f32),
        mesh=_mesh(),
        scratch_types=[
            pltpu.VMEM((TBP,), _f32),
            pltpu.VMEM((TBP,), _f32),
            pltpu.VMEM((CHW,), jnp.int32),
            pltpu.VMEM((CHW,), jnp.int32),
            pltpu.VMEM((CHW,), jnp.int32),
            pltpu.VMEM((CHW,), jnp.int32),
            pltpu.SemaphoreType.DMA,
            pltpu.SemaphoreType.DMA,
            pltpu.SemaphoreType.DMA,
            pltpu.SemaphoreType.DMA,
        ],
        compiler_params=_SC_PARAMS,
    )
    def edge_kernel(g_hbm, as_hbm, ad_hbm, zer_hbm, out,
                    tab, acc, sb0, db0, sb1, db1, s0, s1, s2, s3):
        eh = lax.axis_index("c")
        cg = lax.axis_index("s")
        pltpu.sync_copy(g_hbm.at[cg, 0], tab)
        pltpu.sync_copy(zer_hbm, acc)

        def start(c, sb, db, sems):
            off = pl.multiple_of(c * CHW, 8)
            ca = pltpu.async_copy(as_hbm.at[eh, 0, pl.ds(off, CHW)], sb,
                                  sems[0])
            cb = pltpu.async_copy(ad_hbm.at[eh, 0, pl.ds(off, CHW)], db,
                                  sems[1])
            return ca, cb

        def wait(sb, db, sems):
            pltpu.make_async_copy(as_hbm.at[eh, 0, pl.ds(0, CHW)], sb,
                                  sems[0]).wait()
            pltpu.make_async_copy(ad_hbm.at[eh, 0, pl.ds(0, CHW)], db,
                                  sems[1]).wait()

        def compute(sb, db):
            def vec(i, c):
                for u in range(4):
                    off = (i * 4 + u) * 16
                    a_s = sb[pl.ds(off, 16)]
                    a_d = db[pl.ds(off, 16)]
                    v = plsc.load_gather(tab, [a_s])
                    plsc.addupdate_scatter(acc, [a_d], v)
                return c

            lax.fori_loop(0, VPC // 4, vec, 0)

        start(0, sb0, db0, (s0, s1))

        def outer(c2, c):
            c0 = c2 * 2
            wait(sb0, db0, (s0, s1))
            start(jnp.minimum(c0 + 1, NCH - 1), sb1, db1, (s2, s3))
            compute(sb0, db0)
            wait(sb1, db1, (s2, s3))
            start(jnp.minimum(c0 + 2, NCH - 1), sb0, db0, (s0, s1))
            compute(sb1, db1)
            return c

        lax.fori_loop(0, NCH // 2, outer, 0)
        wait(sb0, db0, (s0, s1))
        pltpu.sync_copy(acc, out.at[cg, eh, 0])

    return edge_kernel(g_all, asrc, adst, zer)


def _dinv_block(degs_ref):
    deg = jnp.sum(degs_ref[:], axis=1)[:, None] + 1.0
    return lax.rsqrt(deg)


def _tc_first(x, W1, degs):
    def body(x_ref, w_ref, d_ref, o_ref):
        dinv = _dinv_block(d_ref)
        o_ref[:] = jnp.dot(x_ref[:], w_ref[:],
                           preferred_element_type=_f32) * dinv

    return pl.pallas_call(
        body,
        grid=(NBK,),
        in_specs=[
            pl.BlockSpec((RB, D), lambda i: (i, 0)),
            pl.BlockSpec((D, H), lambda i: (0, 0)),
            pl.BlockSpec((RB, NW), lambda i: (i, 0)),
        ],
        out_specs=pl.BlockSpec((RB, H), lambda i: (i, 0)),
        out_shape=jax.ShapeDtypeStruct((N, H), _f32),
    )(x, W1, degs)


def _tc_mid(s0, s1, g1, degs, b1, W2):
    def body(s0_ref, s1_ref, g_ref, d_ref, b_ref, w_ref, o_ref):
        dinv = _dinv_block(d_ref)
        a = (s0_ref[:] + s1_ref[:] + g_ref[:]) * dinv + b_ref[:]
        a = jnp.maximum(a, 0.0)
        o_ref[:] = jnp.dot(a, w_ref[:], preferred_element_type=_f32) * dinv

    return pl.pallas_call(
        body,
        grid=(NBK,),
        in_specs=[
            pl.BlockSpec((RB, H), lambda i: (i, 0)),
            pl.BlockSpec((RB, H), lambda i: (i, 0)),
            pl.BlockSpec((RB, H), lambda i: (i, 0)),
            pl.BlockSpec((RB, NW), lambda i: (i, 0)),
            pl.BlockSpec((1, H), lambda i: (0, 0)),
            pl.BlockSpec((H, H), lambda i: (0, 0)),
        ],
        out_specs=pl.BlockSpec((RB, H), lambda i: (i, 0)),
        out_shape=jax.ShapeDtypeStruct((N, H), _f32),
    )(s0, s1, g1, degs, b1, W2)


def _tc_last(s0, s1, g2, degs, b2, bt3, W3, b3):
    def body(s0_ref, s1_ref, g_ref, d_ref, b_ref, bt_ref, w3_ref, b3_ref,
             o_ref, acc_ref):
        i = pl.program_id(0)

        @pl.when(i == 0)
        def _():
            acc_ref[:] = jnp.zeros_like(acc_ref)

        dinv = _dinv_block(d_ref)
        h = (s0_ref[:] + s1_ref[:] + g_ref[:]) * dinv + b_ref[:]
        h = jnp.maximum(h, 0.0)
        seg = bt_ref[0, 0, :]
        onehot = (seg[:, None] == lax.broadcasted_iota(jnp.int32, (1, G), 1)
                  ).astype(_f32)
        acc_ref[:] += lax.dot_general(onehot, h, (((0,), (0,)), ((), ())),
                                      preferred_element_type=_f32)

        @pl.when(i == NBK - 1)
        def _():
            o_ref[:] = (jnp.dot(acc_ref[:], w3_ref[:],
                                preferred_element_type=_f32) + b3_ref[:])

    return pl.pallas_call(
        body,
        grid=(NBK,),
        in_specs=[
            pl.BlockSpec((RB, H), lambda i: (i, 0)),
            pl.BlockSpec((RB, H), lambda i: (i, 0)),
            pl.BlockSpec((RB, H), lambda i: (i, 0)),
            pl.BlockSpec((RB, NW), lambda i: (i, 0)),
            pl.BlockSpec((1, H), lambda i: (0, 0)),
            pl.BlockSpec((1, 1, RB), lambda i: (i, 0, 0)),
            pl.BlockSpec((H, OUT), lambda i: (0, 0)),
            pl.BlockSpec((1, OUT), lambda i: (0, 0)),
        ],
        out_specs=pl.BlockSpec((G, OUT), lambda i: (0, 0)),
        out_shape=jax.ShapeDtypeStruct((G, OUT), _f32),
        scratch_shapes=[pltpu.VMEM((G, H), _f32)],
    )(s0, s1, g2, degs, b2, bt3, W3, b3)


def _pack(g):
    gg = g.reshape(N, CG, CW).transpose(1, 0, 2).reshape(CG, TB)
    return jnp.pad(gg, ((0, 0), (0, TBP - TB))).reshape(CG, 1, TBP)


def _unpack(sacc):
    sr = sacc[:, :, 0, :TB].reshape(CG, 2, N, CW)
    s0 = sr[:, 0].transpose(1, 0, 2).reshape(N, H)
    s1 = sr[:, 1].transpose(1, 0, 2).reshape(N, H)
    return s0, s1


def kernel(x, edge_index, batch, W1, b1, W2, b2, W3, b3):
    src = edge_index[0]
    dst = edge_index[1]
    k4 = jnp.arange(CW, dtype=jnp.int32)

    def _expand(idx):
        a = (idx[:, None] * CW + k4).reshape(2, AW)
        a = jnp.pad(a, ((0, 0), (0, AWP - AW)), constant_values=TB)
        return a.reshape(2, 1, AWP)

    asrc = _expand(src)
    adst = _expand(dst)
    dst2 = jnp.pad(dst.reshape(NW, EPT), ((0, 0), (0, EPTP - EPT)),
                   constant_values=N)
    zn = jnp.zeros((NH,), _f32)
    zer = jnp.zeros((TBP,), _f32)
    bt3 = batch.reshape(NBK, 1, RB)
    b1r = b1.reshape(1, H)
    b2r = b2.reshape(1, H)
    b3r = b3.reshape(1, OUT)

    degs = _sc_degree(dst2, zn)[:, :N].T
    g1 = _tc_first(x, W1, degs)
    s10, s11 = _unpack(_sc_edge(_pack(g1), asrc, adst, zer))
    g2 = _tc_mid(s10, s11, g1, degs, b1r, W2)
    s20, s21 = _unpack(_sc_edge(_pack(g2), asrc, adst, zer))
    return _tc_last(s20, s21, g2, degs, b2r, bt3, W3, b3r)


# speedup vs baseline: 12.7084x; 2.4284x over previous
import functools

import jax
import jax.numpy as jnp
from jax import lax
from jax.experimental import pallas as pl
from jax.experimental.pallas import tpu as pltpu
from jax.experimental.pallas import tpu_sc as plsc

N = 10000
E = 320000
D = 128
H = 64
G = 64
OUT = 1

NC = 2
NS = 16
NW = NC * NS
EPT = E // NW
NH = 10240
EPTP = 10240
CG = 16
CW = H // CG
N2 = 10016
TBP = CW * N2
EHR = E // 2
CHW = 8192
NCH = 20
EHP = NCH * CHW
VPC = CHW // 16

RB = 1000
NBK = N // RB

_f32 = jnp.float32


def _mesh():
    return plsc.VectorSubcoreMesh(core_axis_name="c", subcore_axis_name="s")


_SC_PARAMS = pltpu.CompilerParams(needs_layout_passes=False)


def _sc_degree(dst2, zn):
    @functools.partial(
        pl.kernel,
        out_type=jax.ShapeDtypeStruct((NW, NH), _f32),
        mesh=_mesh(),
        scratch_types=[
            pltpu.VMEM((NH,), _f32),
            pltpu.VMEM((EPTP,), jnp.int32),
        ],
        compiler_params=_SC_PARAMS,
    )
    def deg_kernel(dst_hbm, zn_hbm, out, hist, dv):
        cid = lax.axis_index("c")
        sid = lax.axis_index("s")
        wid = cid * NS + sid
        pltpu.sync_copy(dst_hbm.at[wid], dv)
        pltpu.sync_copy(zn_hbm, hist)
        ones16 = jnp.full((16,), 1.0, _f32)

        def step(i, c):
            for u in range(4):
                idx = dv[pl.ds((i * 4 + u) * 16, 16)]
                plsc.addupdate_scatter(hist, [idx], ones16)
            return c

        lax.fori_loop(0, EPTP // 64, step, 0)
        pltpu.sync_copy(hist, out.at[wid])

    return deg_kernel(dst2, zn)


def _sc_edge(gT, srcp, dstp, zer):
    @functools.partial(
        pl.kernel,
        out_type=jax.ShapeDtypeStruct((CG, 2, 1, TBP), _f32),
        mesh=_mesh(),
        scratch_types=[
            pltpu.VMEM((TBP,), _f32),
            pltpu.VMEM((TBP,), _f32),
            pltpu.VMEM((CHW,), jnp.int32),
            pltpu.VMEM((CHW,), jnp.int32),
            pltpu.VMEM((CHW,), jnp.int32),
            pltpu.VMEM((CHW,), jnp.int32),
            pltpu.SemaphoreType.DMA,
            pltpu.SemaphoreType.DMA,
            pltpu.SemaphoreType.DMA,
            pltpu.SemaphoreType.DMA,
        ],
        compiler_params=_SC_PARAMS,
    )
    def edge_kernel(g_hbm, as_hbm, ad_hbm, zer_hbm, out,
                    tab, acc, sb0, db0, sb1, db1, s0, s1, s2, s3):
        eh = lax.axis_index("c")
        cg = lax.axis_index("s")
        pltpu.sync_copy(g_hbm.at[cg, 0], tab)
        pltpu.sync_copy(zer_hbm, acc)

        def start(c, sb, db, sems):
            off = pl.multiple_of(c * CHW, 128)
            pltpu.async_copy(as_hbm.at[eh, 0, pl.ds(off, CHW)], sb, sems[0])
            pltpu.async_copy(ad_hbm.at[eh, 0, pl.ds(off, CHW)], db, sems[1])

        def wait(sb, db, sems):
            pltpu.make_async_copy(as_hbm.at[eh, 0, pl.ds(0, CHW)], sb,
                                  sems[0]).wait()
            pltpu.make_async_copy(ad_hbm.at[eh, 0, pl.ds(0, CHW)], db,
                                  sems[1]).wait()

        def compute(sb, db):
            def vec(i, c):
                for u in range(2):
                    off = (i * 2 + u) * 16
                    s16 = sb[pl.ds(off, 16)]
                    d16 = db[pl.ds(off, 16)]
                    for k in range(CW):
                        sk = s16 + (k * N2) if k else s16
                        dk = d16 + (k * N2) if k else d16
                        v = plsc.load_gather(tab, [sk])
                        plsc.addupdate_scatter(acc, [dk], v)
                return c

            lax.fori_loop(0, VPC // 2, vec, 0)

        start(0, sb0, db0, (s0, s1))

        def outer(c2, c):
            c0 = c2 * 2
            wait(sb0, db0, (s0, s1))
            start(jnp.minimum(c0 + 1, NCH - 1), sb1, db1, (s2, s3))
            compute(sb0, db0)
            wait(sb1, db1, (s2, s3))
            start(jnp.minimum(c0 + 2, NCH - 1), sb0, db0, (s0, s1))
            compute(sb1, db1)
            return c

        lax.fori_loop(0, NCH // 2, outer, 0)
        wait(sb0, db0, (s0, s1))
        pltpu.sync_copy(acc, out.at[cg, eh, 0])

    return edge_kernel(gT, srcp, dstp, zer)


def _dinv_block(degs_ref):
    deg = jnp.sum(degs_ref[:], axis=1)[:, None] + 1.0
    return lax.rsqrt(deg)


def _tc_first(x, W1, degs):
    def body(x_ref, w_ref, d_ref, o_ref):
        dinv = _dinv_block(d_ref)
        o_ref[:] = jnp.dot(x_ref[:], w_ref[:],
                           preferred_element_type=_f32) * dinv

    return pl.pallas_call(
        body,
        grid=(NBK,),
        in_specs=[
            pl.BlockSpec((RB, D), lambda i: (i, 0)),
            pl.BlockSpec((D, H), lambda i: (0, 0)),
            pl.BlockSpec((RB, NW), lambda i: (i, 0)),
        ],
        out_specs=pl.BlockSpec((RB, H), lambda i: (i, 0)),
        out_shape=jax.ShapeDtypeStruct((N, H), _f32),
    )(x, W1, degs)


def _tc_mid(s0, s1, g1, degs, b1, W2):
    def body(s0_ref, s1_ref, g_ref, d_ref, b_ref, w_ref, o_ref):
        dinv = _dinv_block(d_ref)
        a = (s0_ref[:] + s1_ref[:] + g_ref[:]) * dinv + b_ref[:]
        a = jnp.maximum(a, 0.0)
        o_ref[:] = jnp.dot(a, w_ref[:], preferred_element_type=_f32) * dinv

    return pl.pallas_call(
        body,
        grid=(NBK,),
        in_specs=[
            pl.BlockSpec((RB, H), lambda i: (i, 0)),
            pl.BlockSpec((RB, H), lambda i: (i, 0)),
            pl.BlockSpec((RB, H), lambda i: (i, 0)),
            pl.BlockSpec((RB, NW), lambda i: (i, 0)),
            pl.BlockSpec((1, H), lambda i: (0, 0)),
            pl.BlockSpec((H, H), lambda i: (0, 0)),
        ],
        out_specs=pl.BlockSpec((RB, H), lambda i: (i, 0)),
        out_shape=jax.ShapeDtypeStruct((N, H), _f32),
    )(s0, s1, g1, degs, b1, W2)


def _tc_last(s0, s1, g2, degs, b2, bt3, W3, b3):
    def body(s0_ref, s1_ref, g_ref, d_ref, b_ref, bt_ref, w3_ref, b3_ref,
             o_ref, acc_ref):
        i = pl.program_id(0)

        @pl.when(i == 0)
        def _():
            acc_ref[:] = jnp.zeros_like(acc_ref)

        dinv = _dinv_block(d_ref)
        h = (s0_ref[:] + s1_ref[:] + g_ref[:]) * dinv + b_ref[:]
        h = jnp.maximum(h, 0.0)
        seg = bt_ref[0, 0, :]
        onehot = (seg[:, None] == lax.broadcasted_iota(jnp.int32, (1, G), 1)
                  ).astype(_f32)
        acc_ref[:] += lax.dot_general(onehot, h, (((0,), (0,)), ((), ())),
                                      preferred_element_type=_f32)

        @pl.when(i == NBK - 1)
        def _():
            o_ref[:] = (jnp.dot(acc_ref[:], w3_ref[:],
                                preferred_element_type=_f32) + b3_ref[:])

    return pl.pallas_call(
        body,
        grid=(NBK,),
        in_specs=[
            pl.BlockSpec((RB, H), lambda i: (i, 0)),
            pl.BlockSpec((RB, H), lambda i: (i, 0)),
            pl.BlockSpec((RB, H), lambda i: (i, 0)),
            pl.BlockSpec((RB, NW), lambda i: (i, 0)),
            pl.BlockSpec((1, H), lambda i: (0, 0)),
            pl.BlockSpec((1, 1, RB), lambda i: (i, 0, 0)),
            pl.BlockSpec((H, OUT), lambda i: (0, 0)),
            pl.BlockSpec((1, OUT), lambda i: (0, 0)),
        ],
        out_specs=pl.BlockSpec((G, OUT), lambda i: (0, 0)),
        out_shape=jax.ShapeDtypeStruct((G, OUT), _f32),
        scratch_shapes=[pltpu.VMEM((G, H), _f32)],
    )(s0, s1, g2, degs, b2, bt3, W3, b3)


def _pack(g):
    gt = jnp.pad(g.T, ((0, 0), (0, N2 - N)))
    return gt.reshape(CG, 1, TBP)


def _unpack(sacc):
    sr = sacc[:, :, 0, :].reshape(CG, 2, CW, N2)[:, :, :, :N]
    s0 = sr[:, 0].reshape(H, N).T
    s1 = sr[:, 1].reshape(H, N).T
    return s0, s1


def kernel(x, edge_index, batch, W1, b1, W2, b2, W3, b3):
    src = edge_index[0]
    dst = edge_index[1]

    def _halves(idx):
        a = jnp.pad(idx.reshape(2, EHR), ((0, 0), (0, EHP - EHR)),
                    constant_values=N)
        return a.reshape(2, 1, EHP)

    srcp = _halves(src)
    dstp = _halves(dst)
    dst2 = jnp.pad(dst.reshape(NW, EPT), ((0, 0), (0, EPTP - EPT)),
                   constant_values=N)
    zn = jnp.zeros((NH,), _f32)
    zer = jnp.zeros((TBP,), _f32)
    bt3 = batch.reshape(NBK, 1, RB)
    b1r = b1.reshape(1, H)
    b2r = b2.reshape(1, H)
    b3r = b3.reshape(1, OUT)

    degs = _sc_degree(dst2, zn)[:, :N].T
    g1 = _tc_first(x, W1, degs)
    s10, s11 = _unpack(_sc_edge(_pack(g1), srcp, dstp, zer))
    g2 = _tc_mid(s10, s11, g1, degs, b1r, W2)
    s20, s21 = _unpack(_sc_edge(_pack(g2), srcp, dstp, zer))
    return _tc_last(s20, s21, g2, degs, b2r, bt3, W3, b3r)


# speedup vs baseline: 20.5487x; 1.6169x over previous
import functools

import jax
import jax.numpy as jnp
from jax import lax
from jax.experimental import pallas as pl
from jax.experimental.pallas import tpu as pltpu
from jax.experimental.pallas import tpu_sc as plsc

N = 10000
E = 320000
D = 128
H = 64
G = 64
OUT = 1

NC = 2
NS = 16
NW = NC * NS
EPT = E // NW
NH = 10240
EPTP = 10240
CG = 16
CW = H // CG
N2 = 10016
TBP = CW * N2
EHR = E // 2
CHW = 8192
NCH = 20
EHP = NCH * CHW
VPC = CHW // 16

RB = 1000
NBK = N // RB

_f32 = jnp.float32


def _mesh():
    return plsc.VectorSubcoreMesh(core_axis_name="c", subcore_axis_name="s")


_SC_PARAMS = pltpu.CompilerParams(needs_layout_passes=False)


def _sc_degree(dst2, zn):
    @functools.partial(
        pl.kernel,
        out_type=jax.ShapeDtypeStruct((NW, NH), _f32),
        mesh=_mesh(),
        scratch_types=[
            pltpu.VMEM((NH,), _f32),
            pltpu.VMEM((EPTP,), jnp.int32),
        ],
        compiler_params=_SC_PARAMS,
    )
    def deg_kernel(dst_hbm, zn_hbm, out, hist, dv):
        cid = lax.axis_index("c")
        sid = lax.axis_index("s")
        wid = cid * NS + sid
        pltpu.sync_copy(dst_hbm.at[wid], dv)
        pltpu.sync_copy(zn_hbm, hist)
        ones16 = jnp.full((16,), 1.0, _f32)

        def step(i, c):
            for u in range(4):
                idx = dv[pl.ds((i * 4 + u) * 16, 16)]
                plsc.addupdate_scatter(hist, [idx], ones16)
            return c

        lax.fori_loop(0, EPTP // 64, step, 0)
        pltpu.sync_copy(hist, out.at[wid])

    return deg_kernel(dst2, zn)


def _sc_edge(gT, srcp, dstp, zer):
    @functools.partial(
        pl.kernel,
        out_type=jax.ShapeDtypeStruct((CG, 2, 1, TBP), _f32),
        mesh=_mesh(),
        scratch_types=[
            pltpu.VMEM((TBP,), _f32),
            pltpu.VMEM((TBP,), _f32),
            pltpu.VMEM((CHW,), jnp.int32),
            pltpu.VMEM((CHW,), jnp.int32),
            pltpu.VMEM((CHW,), jnp.int32),
            pltpu.VMEM((CHW,), jnp.int32),
            pltpu.SemaphoreType.DMA,
            pltpu.SemaphoreType.DMA,
            pltpu.SemaphoreType.DMA,
            pltpu.SemaphoreType.DMA,
        ],
        compiler_params=_SC_PARAMS,
    )
    def edge_kernel(g_hbm, as_hbm, ad_hbm, zer_hbm, out,
                    tab, acc, sb0, db0, sb1, db1, s0, s1, s2, s3):
        eh = lax.axis_index("c")
        cg = lax.axis_index("s")
        pltpu.sync_copy(g_hbm.at[cg, 0], tab)
        pltpu.sync_copy(zer_hbm, acc)

        def start(c, sb, db, sems):
            off = pl.multiple_of(c * CHW, 128)
            pltpu.async_copy(as_hbm.at[eh, 0, pl.ds(off, CHW)], sb, sems[0])
            pltpu.async_copy(ad_hbm.at[eh, 0, pl.ds(off, CHW)], db, sems[1])

        def wait(sb, db, sems):
            pltpu.make_async_copy(as_hbm.at[eh, 0, pl.ds(0, CHW)], sb,
                                  sems[0]).wait()
            pltpu.make_async_copy(ad_hbm.at[eh, 0, pl.ds(0, CHW)], db,
                                  sems[1]).wait()

        def compute(sb, db):
            def vec(i, c):
                gathered = []
                for u in range(4):
                    off = (i * 4 + u) * 16
                    s16 = sb[pl.ds(off, 16)]
                    d16 = db[pl.ds(off, 16)]
                    for k in range(CW):
                        sk = s16 + (k * N2) if k else s16
                        dk = d16 + (k * N2) if k else d16
                        gathered.append((dk, plsc.load_gather(tab, [sk])))
                for dk, v in gathered:
                    plsc.addupdate_scatter(acc, [dk], v)
                return c

            lax.fori_loop(0, VPC // 4, vec, 0)

        start(0, sb0, db0, (s0, s1))

        def outer(c2, c):
            c0 = c2 * 2
            wait(sb0, db0, (s0, s1))
            start(jnp.minimum(c0 + 1, NCH - 1), sb1, db1, (s2, s3))
            compute(sb0, db0)
            wait(sb1, db1, (s2, s3))
            start(jnp.minimum(c0 + 2, NCH - 1), sb0, db0, (s0, s1))
            compute(sb1, db1)
            return c

        lax.fori_loop(0, NCH // 2, outer, 0)
        wait(sb0, db0, (s0, s1))
        pltpu.sync_copy(acc, out.at[cg, eh, 0])

    return edge_kernel(gT, srcp, dstp, zer)


def _dinv_block(degs_ref):
    deg = jnp.sum(degs_ref[:], axis=1)[:, None] + 1.0
    return lax.rsqrt(deg)


def _tc_first(x, W1, degs):
    def body(x_ref, w_ref, d_ref, o_ref):
        dinv = _dinv_block(d_ref)
        o_ref[:] = jnp.dot(x_ref[:], w_ref[:],
                           preferred_element_type=_f32) * dinv

    return pl.pallas_call(
        body,
        grid=(NBK,),
        in_specs=[
            pl.BlockSpec((RB, D), lambda i: (i, 0)),
            pl.BlockSpec((D, H), lambda i: (0, 0)),
            pl.BlockSpec((RB, NW), lambda i: (i, 0)),
        ],
        out_specs=pl.BlockSpec((RB, H), lambda i: (i, 0)),
        out_shape=jax.ShapeDtypeStruct((N, H), _f32),
    )(x, W1, degs)


def _tc_mid(s0, s1, g1, degs, b1, W2):
    def body(s0_ref, s1_ref, g_ref, d_ref, b_ref, w_ref, o_ref):
        dinv = _dinv_block(d_ref)
        a = (s0_ref[:] + s1_ref[:] + g_ref[:]) * dinv + b_ref[:]
        a = jnp.maximum(a, 0.0)
        o_ref[:] = jnp.dot(a, w_ref[:], preferred_element_type=_f32) * dinv

    return pl.pallas_call(
        body,
        grid=(NBK,),
        in_specs=[
            pl.BlockSpec((RB, H), lambda i: (i, 0)),
            pl.BlockSpec((RB, H), lambda i: (i, 0)),
            pl.BlockSpec((RB, H), lambda i: (i, 0)),
            pl.BlockSpec((RB, NW), lambda i: (i, 0)),
            pl.BlockSpec((1, H), lambda i: (0, 0)),
            pl.BlockSpec((H, H), lambda i: (0, 0)),
        ],
        out_specs=pl.BlockSpec((RB, H), lambda i: (i, 0)),
        out_shape=jax.ShapeDtypeStruct((N, H), _f32),
    )(s0, s1, g1, degs, b1, W2)


def _tc_last(s0, s1, g2, degs, b2, bt3, W3, b3):
    def body(s0_ref, s1_ref, g_ref, d_ref, b_ref, bt_ref, w3_ref, b3_ref,
             o_ref, acc_ref):
        i = pl.program_id(0)

        @pl.when(i == 0)
        def _():
            acc_ref[:] = jnp.zeros_like(acc_ref)

        dinv = _dinv_block(d_ref)
        h = (s0_ref[:] + s1_ref[:] + g_ref[:]) * dinv + b_ref[:]
        h = jnp.maximum(h, 0.0)
        seg = bt_ref[0, 0, :]
        onehot = (seg[:, None] == lax.broadcasted_iota(jnp.int32, (1, G), 1)
                  ).astype(_f32)
        acc_ref[:] += lax.dot_general(onehot, h, (((0,), (0,)), ((), ())),
                                      preferred_element_type=_f32)

        @pl.when(i == NBK - 1)
        def _():
            o_ref[:] = (jnp.dot(acc_ref[:], w3_ref[:],
                                preferred_element_type=_f32) + b3_ref[:])

    return pl.pallas_call(
        body,
        grid=(NBK,),
        in_specs=[
            pl.BlockSpec((RB, H), lambda i: (i, 0)),
            pl.BlockSpec((RB, H), lambda i: (i, 0)),
            pl.BlockSpec((RB, H), lambda i: (i, 0)),
            pl.BlockSpec((RB, NW), lambda i: (i, 0)),
            pl.BlockSpec((1, H), lambda i: (0, 0)),
            pl.BlockSpec((1, 1, RB), lambda i: (i, 0, 0)),
            pl.BlockSpec((H, OUT), lambda i: (0, 0)),
            pl.BlockSpec((1, OUT), lambda i: (0, 0)),
        ],
        out_specs=pl.BlockSpec((G, OUT), lambda i: (0, 0)),
        out_shape=jax.ShapeDtypeStruct((G, OUT), _f32),
        scratch_shapes=[pltpu.VMEM((G, H), _f32)],
    )(s0, s1, g2, degs, b2, bt3, W3, b3)


def _pack(g):
    gt = jnp.pad(g.T, ((0, 0), (0, N2 - N)))
    return gt.reshape(CG, 1, TBP)


def _unpack(sacc):
    sr = sacc[:, :, 0, :].reshape(CG, 2, CW, N2)[:, :, :, :N]
    s0 = sr[:, 0].reshape(H, N).T
    s1 = sr[:, 1].reshape(H, N).T
    return s0, s1


def kernel(x, edge_index, batch, W1, b1, W2, b2, W3, b3):
    src = edge_index[0]
    dst = edge_index[1]

    def _halves(idx):
        a = jnp.pad(idx.reshape(2, EHR), ((0, 0), (0, EHP - EHR)),
                    constant_values=N)
        return a.reshape(2, 1, EHP)

    srcp = _halves(src)
    dstp = _halves(dst)
    dst2 = jnp.pad(dst.reshape(NW, EPT), ((0, 0), (0, EPTP - EPT)),
                   constant_values=N)
    zn = jnp.zeros((NH,), _f32)
    zer = jnp.zeros((TBP,), _f32)
    bt3 = batch.reshape(NBK, 1, RB)
    b1r = b1.reshape(1, H)
    b2r = b2.reshape(1, H)
    b3r = b3.reshape(1, OUT)

    degs = _sc_degree(dst2, zn)[:, :N].T
    g1 = _tc_first(x, W1, degs)
    s10, s11 = _unpack(_sc_edge(_pack(g1), srcp, dstp, zer))
    g2 = _tc_mid(s10, s11, g1, degs, b1r, W2)
    s20, s21 = _unpack(_sc_edge(_pack(g2), srcp, dstp, zer))
    return _tc_last(s20, s21, g2, degs, b2r, bt3, W3, b3r)


# speedup vs baseline: 21.8459x; 1.0631x over previous
import functools

import jax
import jax.numpy as jnp
from jax import lax
from jax.experimental import pallas as pl
from jax.experimental.pallas import tpu as pltpu
from jax.experimental.pallas import tpu_sc as plsc

N = 10000
E = 320000
D = 128
H = 64
G = 64
OUT = 1

NC = 2
NS = 16
NW = NC * NS
EPT = E // NW
NH = 10240
EPTP = 10240
CG = 16
CW = H // CG
N2 = 10016
TBP = CW * N2
EHR = E // 2
CHW = 8192
NCH = 20
EHP = NCH * CHW
VPC = CHW // 16

RB = 1000
NBK = N // RB

_f32 = jnp.float32


def _mesh():
    return plsc.VectorSubcoreMesh(core_axis_name="c", subcore_axis_name="s")


_SC_PARAMS = pltpu.CompilerParams(needs_layout_passes=False)


def _sc_degree(dst2, zn):
    @functools.partial(
        pl.kernel,
        out_type=jax.ShapeDtypeStruct((NW, NH), _f32),
        mesh=_mesh(),
        scratch_types=[
            pltpu.VMEM((NH,), _f32),
            pltpu.VMEM((EPTP,), jnp.int32),
        ],
        compiler_params=_SC_PARAMS,
    )
    def deg_kernel(dst_hbm, zn_hbm, out, hist, dv):
        cid = lax.axis_index("c")
        sid = lax.axis_index("s")
        wid = cid * NS + sid
        pltpu.sync_copy(dst_hbm.at[wid], dv)
        pltpu.sync_copy(zn_hbm, hist)
        ones16 = jnp.full((16,), 1.0, _f32)

        def step(i, c):
            for u in range(4):
                idx = dv[pl.ds((i * 4 + u) * 16, 16)]
                plsc.addupdate_scatter(hist, [idx], ones16)
            return c

        lax.fori_loop(0, EPTP // 64, step, 0)
        pltpu.sync_copy(hist, out.at[wid])

    return deg_kernel(dst2, zn)


def _sc_edge(gT, srcp, dstp, zer):
    @functools.partial(
        pl.kernel,
        out_type=jax.ShapeDtypeStruct((CG, 2, 1, TBP), _f32),
        mesh=_mesh(),
        scratch_types=[
            pltpu.VMEM((TBP,), _f32),
            pltpu.VMEM((TBP,), _f32),
            pltpu.VMEM((CHW,), jnp.int32),
            pltpu.VMEM((CHW,), jnp.int32),
            pltpu.VMEM((CHW,), jnp.int32),
            pltpu.VMEM((CHW,), jnp.int32),
            pltpu.SemaphoreType.DMA,
            pltpu.SemaphoreType.DMA,
            pltpu.SemaphoreType.DMA,
            pltpu.SemaphoreType.DMA,
        ],
        compiler_params=_SC_PARAMS,
    )
    def edge_kernel(g_hbm, as_hbm, ad_hbm, zer_hbm, out,
                    tab, acc, sb0, db0, sb1, db1, s0, s1, s2, s3):
        eh = lax.axis_index("c")
        cg = lax.axis_index("s")
        pltpu.sync_copy(g_hbm.at[cg, 0], tab)
        pltpu.sync_copy(zer_hbm, acc)

        def start(c, sb, db, sems):
            off = pl.multiple_of(c * CHW, 128)
            pltpu.async_copy(as_hbm.at[eh, 0, pl.ds(off, CHW)], sb, sems[0])
            pltpu.async_copy(ad_hbm.at[eh, 0, pl.ds(off, CHW)], db, sems[1])

        def wait(sb, db, sems):
            pltpu.make_async_copy(as_hbm.at[eh, 0, pl.ds(0, CHW)], sb,
                                  sems[0]).wait()
            pltpu.make_async_copy(ad_hbm.at[eh, 0, pl.ds(0, CHW)], db,
                                  sems[1]).wait()

        def compute(sb, db):
            def vec(i, c):
                gathered = []
                for u in range(4):
                    off = (i * 4 + u) * 16
                    s16 = sb[pl.ds(off, 16)]
                    d16 = db[pl.ds(off, 16)]
                    for k in range(CW):
                        sk = s16 + (k * N2) if k else s16
                        dk = d16 + (k * N2) if k else d16
                        gathered.append((dk, plsc.load_gather(tab, [sk])))
                for dk, v in gathered:
                    plsc.addupdate_scatter(acc, [dk], v)
                return c

            lax.fori_loop(0, VPC // 4, vec, 0)

        start(0, sb0, db0, (s0, s1))

        def outer(c2, c):
            c0 = c2 * 2
            wait(sb0, db0, (s0, s1))
            start(jnp.minimum(c0 + 1, NCH - 1), sb1, db1, (s2, s3))
            compute(sb0, db0)
            wait(sb1, db1, (s2, s3))
            start(jnp.minimum(c0 + 2, NCH - 1), sb0, db0, (s0, s1))
            compute(sb1, db1)
            return c

        lax.fori_loop(0, NCH // 2, outer, 0)
        wait(sb0, db0, (s0, s1))
        pltpu.sync_copy(acc, out.at[cg, eh, 0])

    return edge_kernel(gT, srcp, dstp, zer)


def _dinv_full(degs_ref):
    deg = jnp.sum(degs_ref[:], axis=1)[None, :] + 1.0
    return lax.rsqrt(deg)


def _tc_first(x, W1, degs):
    def body(x_ref, w_ref, d_ref, o_ref):
        dinv = _dinv_full(d_ref)
        g = lax.dot_general(w_ref[:], x_ref[:], (((0,), (1,)), ((), ())),
                            preferred_element_type=_f32)
        o_ref[:] = g * dinv

    return pl.pallas_call(
        body,
        out_shape=jax.ShapeDtypeStruct((H, N), _f32),
    )(x, W1, degs)


def _tc_mid(s0, s1, g1, degs, b1, W2):
    def body(s0_ref, s1_ref, g_ref, d_ref, b_ref, w_ref, o_ref):
        dinv = _dinv_full(d_ref)
        a = (s0_ref[:] + s1_ref[:] + g_ref[:]) * dinv + b_ref[:]
        a = jnp.maximum(a, 0.0)
        g2 = lax.dot_general(w_ref[:], a, (((0,), (0,)), ((), ())),
                             preferred_element_type=_f32)
        o_ref[:] = g2 * dinv

    return pl.pallas_call(
        body,
        out_shape=jax.ShapeDtypeStruct((H, N), _f32),
    )(s0, s1, g1, degs, b1, W2)


def _tc_last(s0, s1, g2, degs, b2, bt2, W3, b3):
    def body(s0_ref, s1_ref, g_ref, d_ref, b_ref, bt_ref, w3_ref, b3_ref,
             o_ref):
        dinv = _dinv_full(d_ref)
        h = (s0_ref[:] + s1_ref[:] + g_ref[:]) * dinv + b_ref[:]
        h = jnp.maximum(h, 0.0)
        seg = bt_ref[0, :]
        onehot = (seg[:, None] == lax.broadcasted_iota(jnp.int32, (1, G), 1)
                  ).astype(_f32)
        poolT = lax.dot_general(h, onehot, (((1,), (0,)), ((), ())),
                                preferred_element_type=_f32)
        o_ref[:] = (lax.dot_general(poolT, w3_ref[:],
                                    (((0,), (0,)), ((), ())),
                                    preferred_element_type=_f32)
                    + b3_ref[:])

    return pl.pallas_call(
        body,
        out_shape=jax.ShapeDtypeStruct((G, OUT), _f32),
    )(s0, s1, g2, degs, b2, bt2, W3, b3)


def _pack(gT):
    return jnp.pad(gT, ((0, 0), (0, N2 - N))).reshape(CG, 1, TBP)


def _unpack(sacc):
    sr = sacc[:, :, 0, :].reshape(CG, 2, CW, N2)
    s0 = sr[:, 0].reshape(H, N2)[:, :N]
    s1 = sr[:, 1].reshape(H, N2)[:, :N]
    return s0, s1


def kernel(x, edge_index, batch, W1, b1, W2, b2, W3, b3):
    src = edge_index[0]
    dst = edge_index[1]

    def _halves(idx):
        a = jnp.pad(idx.reshape(2, EHR), ((0, 0), (0, EHP - EHR)),
                    constant_values=N)
        return a.reshape(2, 1, EHP)

    srcp = _halves(src)
    dstp = _halves(dst)
    dst2 = jnp.pad(dst.reshape(NW, EPT), ((0, 0), (0, EPTP - EPT)),
                   constant_values=N)
    zn = jnp.zeros((NH,), _f32)
    zer = jnp.zeros((TBP,), _f32)
    bt2 = batch.reshape(1, N)
    b1c = b1.reshape(H, 1)
    b2c = b2.reshape(H, 1)
    b3r = b3.reshape(1, OUT)

    degs = _sc_degree(dst2, zn)[:, :N].T
    g1 = _tc_first(x, W1, degs)
    s10, s11 = _unpack(_sc_edge(_pack(g1), srcp, dstp, zer))
    g2 = _tc_mid(s10, s11, g1, degs, b1c, W2)
    s20, s21 = _unpack(_sc_edge(_pack(g2), srcp, dstp, zer))
    return _tc_last(s20, s21, g2, degs, b2c, bt2, W3, b3r)
